# baseline (speedup 1.0000x reference)
import functools

import jax
import jax.numpy as jnp
from jax import lax
from jax.experimental import pallas as pl
from jax.experimental.pallas import tpu as pltpu
from jax.experimental.pallas import tpu_sc as plsc

_GAMMA = 12.0
_BATCH = 16384
_DIM = 32
_NC = 2
_NS = 16
_NW = _NC * _NS
_BPW = _BATCH // _NW
_CHUNK = 128
_NCHUNK = _BPW // _CHUNK


def _sc_body(hi_hbm, ri_hbm, ti_hbm, ent_hbm, rel_hbm, out_hbm,
             idxh, idxr, idxt, rh, rr, rt, outv, sem):
    wid = lax.axis_index("s") * _NC + lax.axis_index("c")
    base = wid * _BPW

    pltpu.sync_copy(hi_hbm.at[wid], idxh)
    pltpu.sync_copy(ri_hbm.at[wid], idxr)
    pltpu.sync_copy(ti_hbm.at[wid], idxt)

    copies = []
    for j in range(_NCHUNK):
        rows = pl.ds(j * _CHUNK, _CHUNK)
        copies.append(pltpu.async_copy(ent_hbm.at[idxh.at[j]], rh.at[rows], sem))
        copies.append(pltpu.async_copy(rel_hbm.at[idxr.at[j]], rr.at[rows], sem))
        copies.append(pltpu.async_copy(ent_hbm.at[idxt.at[j]], rt.at[rows], sem))
    for c in copies:
        c.wait()

    lanes = lax.iota(jnp.int32, 16)

    def group(g, carry):
        rvec = g * 16 + lanes
        acc = jnp.zeros((16,), jnp.float32)
        for j in range(_DIM):
            cvec = jnp.full((16,), j, jnp.int32)
            h = plsc.load_gather(rh, [rvec, cvec])
            r = plsc.load_gather(rr, [rvec, cvec])
            t = plsc.load_gather(rt, [rvec, cvec])
            acc = acc + jnp.abs(h + r - t)
        outv[pl.ds(g * 16, 16)] = _GAMMA - acc
        return carry

    lax.fori_loop(0, _BPW // 16, group, 0)

    pltpu.sync_copy(outv, out_hbm.at[pl.ds(base, _BPW)])


_mesh = plsc.VectorSubcoreMesh(core_axis_name="c", subcore_axis_name="s")

_sc_kernel = functools.partial(
    pl.kernel,
    mesh=_mesh,
    out_type=jax.ShapeDtypeStruct((_BATCH,), jnp.float32),
    scratch_types=[
        pltpu.VMEM((_NCHUNK, _CHUNK), jnp.int32),
        pltpu.VMEM((_NCHUNK, _CHUNK), jnp.int32),
        pltpu.VMEM((_NCHUNK, _CHUNK), jnp.int32),
        pltpu.VMEM((_BPW, _DIM), jnp.float32),
        pltpu.VMEM((_BPW, _DIM), jnp.float32),
        pltpu.VMEM((_BPW, _DIM), jnp.float32),
        pltpu.VMEM((_BPW,), jnp.float32),
        pltpu.SemaphoreType.DMA,
    ],
    compiler_params=pltpu.CompilerParams(
        needs_layout_passes=False, use_tc_tiling_on_sc=False),
)(_sc_body)


def kernel(sample, entity_embedding, relation_embedding):
    heads = sample[:, 0].reshape(_NW, _NCHUNK, _CHUNK)
    rels = sample[:, 1].reshape(_NW, _NCHUNK, _CHUNK)
    tails = sample[:, 2].reshape(_NW, _NCHUNK, _CHUNK)
    score = _sc_kernel(heads, rels, tails, entity_embedding, relation_embedding)
    return score.reshape(_BATCH, 1)
